# baseline (speedup 1.0000x reference)
import jax
import jax.numpy as jnp
from jax import lax
from jax.experimental import pallas as pl
from jax.experimental.pallas import tpu as pltpu
from jax.experimental.pallas import tpu_sc as plsc

_B, _T, _D = 64, 1025, 768
_NC, _NS, _L = 2, 16, 16
_NW = _NC * _NS
_TPW = (_T + _NW - 1) // _NW
_NJ = _D // _L


def _sc_kernel(x_hbm, pos_hbm, fm_hbm, me_hbm, out_hbm,
               xbuf, obuf, posrow, merow, fmbuf):
    wid = lax.axis_index("s") * _NC + lax.axis_index("c")
    pltpu.sync_copy(me_hbm.at[0], merow)
    t0 = wid * _TPW

    def token_body(i, carry):
        t = t0 + i

        @pl.when(t < _T)
        def _():
            pltpu.sync_copy(x_hbm.at[t], xbuf)
            pltpu.sync_copy(pos_hbm.at[t, 0], posrow)
            pltpu.sync_copy(fm_hbm.at[t], fmbuf)

            def j_body(j, carry2):
                pj = posrow[pl.ds(j * _L, _L)]
                mj = pj + merow[pl.ds(j * _L, _L)]

                def b_body(b, carry3):
                    kv = fmbuf[b, :]
                    v = xbuf[b, pl.ds(j * _L, _L)]
                    obuf[b, pl.ds(j * _L, _L)] = jnp.where(kv != 0, mj, v + pj)
                    return carry3

                lax.fori_loop(0, _B, b_body, 0)
                return carry2

            lax.fori_loop(0, _NJ, j_body, 0)
            pltpu.sync_copy(obuf, out_hbm.at[t])

        return carry

    lax.fori_loop(0, _TPW, token_body, 0)


def kernel(x, pos_embed, mask, masked_embed):
    B, T, D = x.shape
    xt = jnp.transpose(x, (1, 0, 2))
    post = jnp.transpose(pos_embed, (1, 0, 2))
    mt = mask.reshape(B, T - 1).T.astype(jnp.int32)
    fm = jnp.pad(mt, ((1, 0), (0, 0)))
    fm_exp = jnp.broadcast_to(fm[:, :, None], (T, B, _L))

    mesh = plsc.VectorSubcoreMesh(core_axis_name="c", subcore_axis_name="s")
    out_t = pl.kernel(
        _sc_kernel,
        mesh=mesh,
        out_type=jax.ShapeDtypeStruct((T, B, D), x.dtype),
        scratch_types=[
            pltpu.VMEM((B, D), jnp.float32),
            pltpu.VMEM((B, D), jnp.float32),
            pltpu.VMEM((D,), jnp.float32),
            pltpu.VMEM((D,), jnp.float32),
            pltpu.VMEM((B, _L), jnp.int32),
        ],
    )(xt, post, fm_exp, masked_embed)
    return jnp.transpose(out_t, (1, 0, 2))


# speedup vs baseline: 7.3075x; 7.3075x over previous
import jax
import jax.numpy as jnp
from jax.experimental import pallas as pl

_TB = 41


def _select_add_kernel(x_ref, pos_ref, fm_ref, me_ref, o_ref):
    xv = x_ref[...]
    me = me_ref[...][None]
    fm = jnp.transpose(fm_ref[...], (0, 2, 1))
    o_ref[...] = jnp.where(fm, me, xv) + pos_ref[...]


def kernel(x, pos_embed, mask, masked_embed):
    B, T, D = x.shape
    xt = jnp.transpose(x, (1, 0, 2))
    post = jnp.transpose(pos_embed, (1, 0, 2))
    mt = mask.transpose(1, 2, 0).reshape(T - 1, B)
    fm = jnp.pad(mt, ((1, 0), (0, 0))).reshape(T, 1, B)

    out_t = pl.pallas_call(
        _select_add_kernel,
        grid=(T // _TB,),
        in_specs=[
            pl.BlockSpec((_TB, B, D), lambda t: (t, 0, 0)),
            pl.BlockSpec((_TB, 1, D), lambda t: (t, 0, 0)),
            pl.BlockSpec((_TB, 1, B), lambda t: (t, 0, 0)),
            pl.BlockSpec((1, D), lambda t: (0, 0)),
        ],
        out_specs=pl.BlockSpec((_TB, B, D), lambda t: (t, 0, 0)),
        out_shape=jax.ShapeDtypeStruct((T, B, D), x.dtype),
    )(xt, post, fm, masked_embed)
    return jnp.transpose(out_t, (1, 0, 2))
